# Initial kernel scaffold; baseline (speedup 1.0000x reference)
import jax
import jax.numpy as jnp
from jax.experimental import pallas as pl


def kernel(x_srna, x_mrna, edge_src, edge_dst, label_src, label_dst, W1t_l, b1t_l, W1t_r, W1r_l, b1r_l, W1r_r, W2t_l, b2t_l, W2t_r, W2r_l, b2r_l, W2r_r):
    raise NotImplementedError("write your pallas kernel here")


# speedup vs baseline: 1.0368x; 1.0368x over previous
import functools

import jax
import jax.numpy as jnp
from jax.experimental import pallas as pl
from jax.experimental.pallas import tpu as pltpu

N_SRNA_ = 10000
N_MRNA_ = 50000
E_ = 600000
L_ = 100000
D_ = 128


def _combine_body(relu, s_ref, c_ref, x_ref, wl_ref, b_ref, wr_ref, o_ref):
    c = c_ref[:, 0:1]
    mean = s_ref[...] / jnp.maximum(c, 1.0)
    acc = jnp.dot(mean, wl_ref[...], preferred_element_type=jnp.float32)
    acc = acc + b_ref[...]
    acc = acc + jnp.dot(x_ref[...], wr_ref[...], preferred_element_type=jnp.float32)
    o_ref[...] = jnp.maximum(acc, 0.0) if relu else acc


def _combine(sums, cnt16, x, W_l, b_l, W_r, relu):
    N = x.shape[0]
    BS = 1000
    b2d = b_l.reshape(1, D_)
    return pl.pallas_call(
        functools.partial(_combine_body, relu),
        grid=(N // BS,),
        in_specs=[
            pl.BlockSpec((BS, D_), lambda i: (i, 0)),
            pl.BlockSpec((BS, 16), lambda i: (i, 0)),
            pl.BlockSpec((BS, D_), lambda i: (i, 0)),
            pl.BlockSpec((D_, D_), lambda i: (0, 0)),
            pl.BlockSpec((1, D_), lambda i: (0, 0)),
            pl.BlockSpec((D_, D_), lambda i: (0, 0)),
        ],
        out_specs=pl.BlockSpec((BS, D_), lambda i: (i, 0)),
        out_shape=jax.ShapeDtypeStruct((N, D_), jnp.float32),
    )(sums, cnt16, x, W_l, b2d, W_r)


def _decoder_body(a_ref, b_ref, o_ref):
    o_ref[...] = jnp.sum(a_ref[...] * b_ref[...], axis=1, keepdims=True)


def _decoder(a, b):
    N = a.shape[0]
    BS = 1000
    out = pl.pallas_call(
        _decoder_body,
        grid=(N // BS,),
        in_specs=[
            pl.BlockSpec((BS, D_), lambda i: (i, 0)),
            pl.BlockSpec((BS, D_), lambda i: (i, 0)),
        ],
        out_specs=pl.BlockSpec((BS, 1), lambda i: (i, 0)),
        out_shape=jax.ShapeDtypeStruct((N, 1), jnp.float32),
    )(a, b)
    return out.reshape(-1)


def _seg_sums(table, idx_g, idx_s, n_dst):
    msgs = table[idx_g]
    sums = jax.ops.segment_sum(msgs, idx_s, num_segments=n_dst)
    cnt = jax.ops.segment_sum(
        jnp.ones((idx_g.shape[0], 1), jnp.float32), idx_s, num_segments=n_dst)
    cnt16 = jnp.pad(cnt, ((0, 0), (0, 15)))
    return sums, cnt16


def kernel(x_srna, x_mrna, edge_src, edge_dst, label_src, label_dst,
           W1t_l, b1t_l, W1t_r, W1r_l, b1r_l, W1r_r,
           W2t_l, b2t_l, W2t_r, W2r_l, b2r_l, W2r_r):
    edge_src = edge_src.astype(jnp.int32)
    edge_dst = edge_dst.astype(jnp.int32)
    label_src = label_src.astype(jnp.int32)
    label_dst = label_dst.astype(jnp.int32)

    sums_m, cnt_m = _seg_sums(x_srna, edge_src, edge_dst, N_MRNA_)
    sums_s, cnt_s = _seg_sums(x_mrna, edge_dst, edge_src, N_SRNA_)
    z_mrna = _combine(sums_m, cnt_m, x_mrna, W1t_l, b1t_l, W1t_r, True)
    z_srna = _combine(sums_s, cnt_s, x_srna, W1r_l, b1r_l, W1r_r, True)

    sums_m2, _ = _seg_sums(z_srna, edge_src, edge_dst, N_MRNA_)
    sums_s2, _ = _seg_sums(z_mrna, edge_dst, edge_src, N_SRNA_)
    z_mrna2 = _combine(sums_m2, cnt_m, z_mrna, W2t_l, b2t_l, W2t_r, False)
    z_srna2 = _combine(sums_s2, cnt_s, z_srna, W2r_l, b2r_l, W2r_r, False)

    return _decoder(z_srna2[label_src], z_mrna2[label_dst])


# speedup vs baseline: 1.0471x; 1.0099x over previous
import functools

import jax
import jax.numpy as jnp
from jax import lax
from jax.experimental import pallas as pl
from jax.experimental.pallas import tpu as pltpu
from jax.experimental.pallas import tpu_sc as plsc

N_SRNA_ = 10000
N_MRNA_ = 50000
E_ = 600000
L_ = 100000
D_ = 128

NT_ = 16
L_PAD_ = 102400

_MESH = plsc.VectorSubcoreMesh(core_axis_name="c", subcore_axis_name="s")


def _sc_decoder(za, zb, ls, ld):
    lp = ls.shape[0]
    pt = lp // (2 * NT_)
    BT = 128
    nb = pt // BT

    scratch = [
        pltpu.VMEM((pt,), jnp.int32),
        pltpu.VMEM((pt,), jnp.int32),
        pltpu.VMEM((BT, D_), jnp.float32),
        pltpu.VMEM((BT, D_), jnp.float32),
        pltpu.VMEM((BT, D_), jnp.float32),
        pltpu.VMEM((BT, D_), jnp.float32),
        pltpu.VMEM((pt,), jnp.float32),
        pltpu.SemaphoreType.DMA,
        pltpu.SemaphoreType.DMA,
    ]

    def body(za_h, zb_h, ls_h, ld_h, out_h,
             lsv, ldv, ra0, ra1, rb0, rb1, outv, sema, semb):
        c = lax.axis_index("c")
        s = lax.axis_index("s")
        base = (c * NT_ + s) * pt
        pltpu.sync_copy(ls_h.at[pl.ds(base, pt)], lsv)
        pltpu.sync_copy(ld_h.at[pl.ds(base, pt)], ldv)
        ra = (ra0, ra1)
        rb = (rb0, rb1)

        def ga(k, buf):
            return pltpu.async_copy(za_h.at[lsv.at[pl.ds(k * BT, BT)]], buf, sema)

        def gb(k, buf):
            return pltpu.async_copy(zb_h.at[ldv.at[pl.ds(k * BT, BT)]], buf, semb)

        da, db = ga(0, ra0), gb(0, rb0)
        for k in range(nb):
            if k + 1 < nb:
                da2, db2 = ga(k + 1, ra[(k + 1) % 2]), gb(k + 1, rb[(k + 1) % 2])
            da.wait()
            db.wait()
            a_buf, b_buf = ra[k % 2], rb[k % 2]
            lanes = lax.iota(jnp.int32, 16)
            perms = [jnp.bitwise_xor(lanes, sh) for sh in (8, 4, 2, 1)]

            def lab16(g, _):
                def one(jj, res):
                    j = g * 16 + jj
                    acc = a_buf[j, pl.ds(0, 16)] * b_buf[j, pl.ds(0, 16)]
                    for t in range(1, D_ // 16):
                        acc = acc + a_buf[j, pl.ds(t * 16, 16)] * b_buf[j, pl.ds(t * 16, 16)]
                    for p in perms:
                        acc = acc + acc.at[p].get(mode="promise_in_bounds")
                    return jnp.where(lanes == jj, acc, res)

                res = lax.fori_loop(0, 16, one, jnp.zeros((16,), jnp.float32))
                outv[pl.ds(k * BT + g * 16, 16)] = res
                return 0

            lax.fori_loop(0, BT // 16, lab16, 0)
            if k + 1 < nb:
                da, db = da2, db2
        pltpu.sync_copy(outv, out_h.at[pl.ds(base, pt)])

    f = pl.kernel(body, out_type=jax.ShapeDtypeStruct((lp,), jnp.float32),
                  mesh=_MESH, scratch_types=tuple(scratch))
    return f(za, zb, ls, ld)


def _combine_body(relu, s_ref, c_ref, x_ref, wl_ref, b_ref, wr_ref, o_ref):
    cval = c_ref[:, 0:1]
    mean = s_ref[...] / jnp.maximum(cval, 1.0)
    acc = jnp.dot(mean, wl_ref[...], preferred_element_type=jnp.float32)
    acc = acc + b_ref[...]
    acc = acc + jnp.dot(x_ref[...], wr_ref[...], preferred_element_type=jnp.float32)
    o_ref[...] = jnp.maximum(acc, 0.0) if relu else acc


def _combine(sums, cnt, x, W_l, b_l, W_r, relu):
    N = x.shape[0]
    BS = 1000
    b2d = b_l.reshape(1, D_)
    return pl.pallas_call(
        functools.partial(_combine_body, relu),
        grid=(N // BS,),
        in_specs=[
            pl.BlockSpec((BS, D_), lambda i: (i, 0)),
            pl.BlockSpec((BS, 1), lambda i: (i, 0)),
            pl.BlockSpec((BS, D_), lambda i: (i, 0)),
            pl.BlockSpec((D_, D_), lambda i: (0, 0)),
            pl.BlockSpec((1, D_), lambda i: (0, 0)),
            pl.BlockSpec((D_, D_), lambda i: (0, 0)),
        ],
        out_specs=pl.BlockSpec((BS, D_), lambda i: (i, 0)),
        out_shape=jax.ShapeDtypeStruct((N, D_), jnp.float32),
    )(sums, cnt, x, W_l, b2d, W_r)


def _seg_sums(table, idx_g, idx_s, n_dst):
    msgs = table[idx_g]
    return jax.ops.segment_sum(msgs, idx_s, num_segments=n_dst)


def kernel(x_srna, x_mrna, edge_src, edge_dst, label_src, label_dst,
           W1t_l, b1t_l, W1t_r, W1r_l, b1r_l, W1r_r,
           W2t_l, b2t_l, W2t_r, W2r_l, b2r_l, W2r_r):
    edge_src = edge_src.astype(jnp.int32)
    edge_dst = edge_dst.astype(jnp.int32)
    label_src = label_src.astype(jnp.int32)
    label_dst = label_dst.astype(jnp.int32)

    ones_e = jnp.ones((E_, 1), jnp.float32)
    cnt_m = jax.ops.segment_sum(ones_e, edge_dst, num_segments=N_MRNA_)
    cnt_s = jax.ops.segment_sum(ones_e, edge_src, num_segments=N_SRNA_)

    sums_m = _seg_sums(x_srna, edge_src, edge_dst, N_MRNA_)
    sums_s = _seg_sums(x_mrna, edge_dst, edge_src, N_SRNA_)
    z_mrna = _combine(sums_m, cnt_m, x_mrna, W1t_l, b1t_l, W1t_r, True)
    z_srna = _combine(sums_s, cnt_s, x_srna, W1r_l, b1r_l, W1r_r, True)

    sums_m2 = _seg_sums(z_srna, edge_src, edge_dst, N_MRNA_)
    sums_s2 = _seg_sums(z_mrna, edge_dst, edge_src, N_SRNA_)
    z_mrna2 = _combine(sums_m2, cnt_m, z_mrna, W2t_l, b2t_l, W2t_r, False)
    z_srna2 = _combine(sums_s2, cnt_s, z_srna, W2r_l, b2r_l, W2r_r, False)

    lpad = jnp.zeros((L_PAD_ - L_,), jnp.int32)
    ls = jnp.concatenate([label_src, lpad])
    ld = jnp.concatenate([label_dst, lpad])
    return _sc_decoder(z_srna2, z_mrna2, ls, ld)[:L_]
